# baseline (speedup 1.0000x reference)
import functools

import jax
import jax.numpy as jnp
from jax import lax
from jax.experimental import pallas as pl
from jax.experimental.pallas import tpu as pltpu
from jax.experimental.pallas import tpu_sc as plsc

BATCH = 16384
EMBED_DIM = 32
NUM_CORES = 2
NUM_SUBCORES = 16
NUM_WORKERS = NUM_CORES * NUM_SUBCORES
B_PER_W = BATCH // NUM_WORKERS
LANES = 16
GSIZE = 4
NSLOT = 3
N_GROUPS = B_PER_W // GSIZE
IDX_PAD = B_PER_W + LANES


def _gather_body(su_hbm, fu_hbm, bu_hbm, pu_hbm,
                 si_hbm, fi_hbm, bi_hbm, pi_hbm,
                 ut_hbm, it_hbm, uembf_hbm, iembf_hbm,
                 suv, fuv, buv, puv, siv, fiv, biv, piv,
                 ubuf, ibuf, ustage, istage,
                 semu0, semu1, semu2, semi0, semi1, semi2,
                 osemu0, osemu1, osemu2, osemi0, osemi1, osemi2):
    wid = lax.axis_index("s") * NUM_CORES + lax.axis_index("c")
    base = wid * B_PER_W

    for src, dst in ((su_hbm, suv), (fu_hbm, fuv), (bu_hbm, buv),
                     (pu_hbm, puv), (si_hbm, siv), (fi_hbm, fiv),
                     (bi_hbm, biv), (pi_hbm, piv)):
        pltpu.sync_copy(src.at[pl.ds(base, B_PER_W)],
                        dst.at[pl.ds(0, B_PER_W)])

    semus = (semu0, semu1, semu2)
    semis = (semi0, semi1, semi2)
    osemus = (osemu0, osemu1, osemu2)
    osemis = (osemi0, osemi1, osemi2)
    rows0 = lax.iota(jnp.int32, LANES)
    rows1 = rows0 + LANES

    def issue(g, slot):
        for sv, fv, buf, sems in ((suv, fuv, ubuf, semus),
                                  (siv, fiv, ibuf, semis)):
            svec = sv[pl.ds(g * GSIZE, LANES)]
            fvec = fv[pl.ds(g * GSIZE, LANES)]
            for k in range(GSIZE):
                @pl.when(fvec[k] != 0)
                def _(svec=svec, k=k, buf=buf, sems=sems):
                    off = pl.multiple_of((svec[k] >> 7) << 7, 128)
                    tab = ut_hbm if buf is ubuf else it_hbm
                    pltpu.async_copy(tab.at[:, pl.ds(off, 128)],
                                     buf.at[slot, k], sems[slot])

    def drain(g, slot):
        for fv, buf, sems in ((fuv, ubuf, semus), (fiv, ibuf, semis)):
            fvec = fv[pl.ds(g * GSIZE, LANES)]
            for k in range(GSIZE):
                @pl.when(fvec[k] != 0)
                def _(k=k, buf=buf, sems=sems):
                    tab = ut_hbm if buf is ubuf else it_hbm
                    pltpu.make_async_copy(tab.at[:, pl.ds(0, 128)],
                                          buf.at[slot, k],
                                          sems[slot]).wait()

    def drain_out(g, slot):
        @pl.when(g >= NSLOT)
        def _():
            for stage, emb, osems in ((ustage, uembf_hbm, osemus),
                                      (istage, iembf_hbm, osemis)):
                for k in range(GSIZE):
                    pltpu.make_async_copy(stage.at[slot, k],
                                          emb.at[pl.ds(0, EMBED_DIM)],
                                          osems[slot]).wait()

    def compute(g, slot):
        for sv, bv, pv, buf, stage, emb, osems in (
                (suv, buv, puv, ubuf, ustage, uembf_hbm, osemus),
                (siv, biv, piv, ibuf, istage, iembf_hbm, osemis)):
            svec = sv[pl.ds(g * GSIZE, LANES)]
            bvec = bv[pl.ds(g * GSIZE, LANES)]
            pvec = pv[pl.ds(g * GSIZE, LANES)]
            cvec = svec & 127
            for k in range(GSIZE):
                ref = buf.at[slot, bvec[k]]
                c = jnp.full((LANES,), cvec[k], jnp.int32)
                e0 = plsc.load_gather(ref, [rows0, c])
                e1 = plsc.load_gather(ref, [rows1, c])
                stage[slot, k, pl.ds(0, LANES)] = e0
                stage[slot, k, pl.ds(LANES, LANES)] = e1
                pltpu.async_copy(stage.at[slot, k],
                                 emb.at[pl.ds(pvec[k] * EMBED_DIM,
                                              EMBED_DIM)],
                                 osems[slot])

    issue(jnp.int32(0), 0)
    issue(jnp.int32(1), 1)
    issue(jnp.int32(2), 2)

    def superstep(t, carry):
        for j in range(NSLOT):
            g = NSLOT * t + j
            drain(g, j)
            drain_out(g, j)
            compute(g, j)

            @pl.when(g < N_GROUPS - NSLOT)
            def _(j=j, g=g):
                issue(g + NSLOT, j)

        return carry

    n_steps = (N_GROUPS - 2) // NSLOT
    lax.fori_loop(0, n_steps, superstep, None)
    for g in range(NSLOT * n_steps, N_GROUPS):
        drain(jnp.int32(g), g % NSLOT)
        drain_out(jnp.int32(g), g % NSLOT)
        compute(jnp.int32(g), g % NSLOT)
    for slot in range(NSLOT):
        for stage, emb, osems in ((ustage, uembf_hbm, osemus),
                                  (istage, iembf_hbm, osemis)):
            for k in range(GSIZE):
                pltpu.make_async_copy(stage.at[slot, k],
                                      emb.at[pl.ds(0, EMBED_DIM)],
                                      osems[slot]).wait()


def _dot_body(uembf_hbm, iembf_hbm, out_hbm, uv, iv, acc_v):
    wid = lax.axis_index("s") * NUM_CORES + lax.axis_index("c")
    base = wid * B_PER_W

    pltpu.sync_copy(uembf_hbm.at[pl.ds(base * EMBED_DIM,
                                       B_PER_W * EMBED_DIM)], uv)
    pltpu.sync_copy(iembf_hbm.at[pl.ds(base * EMBED_DIM,
                                       B_PER_W * EMBED_DIM)], iv)

    lane = lax.iota(jnp.int32, LANES)

    def chunk(c, carry):
        accs = jnp.zeros((LANES,), jnp.float32)
        for r in range(LANES):
            rb = (c * LANES + r) * EMBED_DIM
            p = (uv[pl.ds(rb, LANES)] * iv[pl.ds(rb, LANES)]
                 + uv[pl.ds(rb + LANES, LANES)]
                 * iv[pl.ds(rb + LANES, LANES)])
            s = jnp.sum(p)
            accs = jnp.where(lane == r, jnp.broadcast_to(s, (LANES,)), accs)
        acc_v[pl.ds(c * LANES, LANES)] = accs
        return carry

    lax.fori_loop(0, B_PER_W // LANES, chunk, None)
    pltpu.sync_copy(acc_v, out_hbm.at[pl.ds(base, B_PER_W)])


def _prep(user_ids, item_ids):
    iot = lax.iota(jnp.int32, BATCH)
    keys = jnp.stack([user_ids, item_ids])
    vals = jnp.broadcast_to(iot, (2, BATCH))
    s2, p2 = lax.sort_key_val(keys, vals, dimension=1)
    q2 = s2 >> 7
    newcol = jnp.concatenate(
        [jnp.ones((2, 1), jnp.int32),
         (q2[:, 1:] != q2[:, :-1]).astype(jnp.int32)], axis=1)
    flag2 = newcol | jnp.broadcast_to(
        ((iot & (GSIZE - 1)) == 0).astype(jnp.int32), (2, BATCH))
    pos = iot & (GSIZE - 1)
    bidx2 = lax.cummax(
        jnp.where(flag2 != 0, pos, 0).reshape(2, -1, GSIZE),
        axis=2).reshape(2, BATCH).astype(jnp.int32)
    return s2, flag2, bidx2, p2


@jax.jit
def kernel(user_ids, item_ids, user_table, item_table):
    s2, f2, b2, p2 = _prep(user_ids.astype(jnp.int32),
                           item_ids.astype(jnp.int32))
    su, si = s2[0], s2[1]
    fu, fi = f2[0], f2[1]
    bu, bi = b2[0], b2[1]
    pu, pi = p2[0], p2[1]

    mesh = plsc.VectorSubcoreMesh(core_axis_name="c", subcore_axis_name="s")
    params = pltpu.CompilerParams(needs_layout_passes=False)

    gather_k = functools.partial(
        pl.kernel,
        mesh=mesh,
        compiler_params=params,
        out_type=(
            jax.ShapeDtypeStruct((BATCH * EMBED_DIM,), jnp.float32),
            jax.ShapeDtypeStruct((BATCH * EMBED_DIM,), jnp.float32),
        ),
        scratch_types=(
            [pltpu.VMEM((IDX_PAD,), jnp.int32) for _ in range(8)]
            + [
                pltpu.VMEM((NSLOT, GSIZE, EMBED_DIM, 128), jnp.float32),
                pltpu.VMEM((NSLOT, GSIZE, EMBED_DIM, 128), jnp.float32),
                pltpu.VMEM((NSLOT, GSIZE, EMBED_DIM), jnp.float32),
                pltpu.VMEM((NSLOT, GSIZE, EMBED_DIM), jnp.float32),
            ]
            + [pltpu.SemaphoreType.DMA for _ in range(12)]
        ),
    )(_gather_body)
    uembf, iembf = gather_k(su, fu, bu, pu, si, fi, bi, pi,
                            user_table.T, item_table.T)

    dot_k = functools.partial(
        pl.kernel,
        mesh=mesh,
        compiler_params=params,
        out_type=jax.ShapeDtypeStruct((BATCH,), jnp.float32),
        scratch_types=[
            pltpu.VMEM((B_PER_W * EMBED_DIM,), jnp.float32),
            pltpu.VMEM((B_PER_W * EMBED_DIM,), jnp.float32),
            pltpu.VMEM((B_PER_W,), jnp.float32),
        ],
    )(_dot_body)
    return dot_k(uembf, iembf)


# speedup vs baseline: 1.3088x; 1.3088x over previous
import functools

import jax
import jax.numpy as jnp
from jax import lax
from jax.experimental import pallas as pl
from jax.experimental.pallas import tpu as pltpu
from jax.experimental.pallas import tpu_sc as plsc

BATCH = 16384
EMBED_DIM = 32
NUM_CORES = 2
NUM_SUBCORES = 16
NUM_WORKERS = NUM_CORES * NUM_SUBCORES
B_PER_W = BATCH // NUM_WORKERS
LANES = 16
GSIZE = 4
NSLOT = 3
N_GROUPS = B_PER_W // GSIZE
IDX_PAD = B_PER_W + LANES


def _gather_body(su_hbm, fu_hbm, bu_hbm, pu_hbm,
                 si_hbm, fi_hbm, bi_hbm, pi_hbm,
                 ut_hbm, it_hbm, uembf_hbm, iembf_hbm,
                 suv, fuv, buv, puv, siv, fiv, biv, piv,
                 ubuf, ibuf, ustage, istage,
                 semu0, semu1, semu2, semi0, semi1, semi2,
                 osemu0, osemu1, osemu2, osemi0, osemi1, osemi2):
    wid = lax.axis_index("s") * NUM_CORES + lax.axis_index("c")
    base = wid * B_PER_W

    for src, dst in ((su_hbm, suv), (fu_hbm, fuv), (bu_hbm, buv),
                     (pu_hbm, puv), (si_hbm, siv), (fi_hbm, fiv),
                     (bi_hbm, biv), (pi_hbm, piv)):
        pltpu.sync_copy(src.at[pl.ds(base, B_PER_W)],
                        dst.at[pl.ds(0, B_PER_W)])

    semus = (semu0, semu1, semu2)
    semis = (semi0, semi1, semi2)
    osemus = (osemu0, osemu1, osemu2)
    osemis = (osemi0, osemi1, osemi2)
    rows0 = lax.iota(jnp.int32, LANES)
    rows1 = rows0 + LANES

    def issue(g, slot):
        for sv, fv, buf, sems in ((suv, fuv, ubuf, semus),
                                  (siv, fiv, ibuf, semis)):
            svec = sv[pl.ds(g * GSIZE, LANES)]
            fvec = fv[pl.ds(g * GSIZE, LANES)]
            for k in range(GSIZE):
                @pl.when(fvec[k] != 0)
                def _(svec=svec, k=k, buf=buf, sems=sems):
                    off = pl.multiple_of((svec[k] >> 7) << 7, 128)
                    tab = ut_hbm if buf is ubuf else it_hbm
                    pltpu.async_copy(tab.at[:, pl.ds(off, 128)],
                                     buf.at[slot, k], sems[slot])

    def drain(g, slot):
        for fv, buf, sems in ((fuv, ubuf, semus), (fiv, ibuf, semis)):
            fvec = fv[pl.ds(g * GSIZE, LANES)]
            for k in range(GSIZE):
                @pl.when(fvec[k] != 0)
                def _(k=k, buf=buf, sems=sems):
                    tab = ut_hbm if buf is ubuf else it_hbm
                    pltpu.make_async_copy(tab.at[:, pl.ds(0, 128)],
                                          buf.at[slot, k],
                                          sems[slot]).wait()

    def drain_out(g, slot):
        @pl.when(g >= NSLOT)
        def _():
            for stage, emb, osems in ((ustage, uembf_hbm, osemus),
                                      (istage, iembf_hbm, osemis)):
                for k in range(GSIZE):
                    pltpu.make_async_copy(stage.at[slot, k],
                                          emb.at[pl.ds(0, EMBED_DIM)],
                                          osems[slot]).wait()

    def compute(g, slot):
        for sv, bv, pv, buf, stage, emb, osems in (
                (suv, buv, puv, ubuf, ustage, uembf_hbm, osemus),
                (siv, biv, piv, ibuf, istage, iembf_hbm, osemis)):
            svec = sv[pl.ds(g * GSIZE, LANES)]
            bvec = bv[pl.ds(g * GSIZE, LANES)]
            pvec = pv[pl.ds(g * GSIZE, LANES)]
            cvec = svec & 127
            for k in range(GSIZE):
                ref = buf.at[slot, bvec[k]]
                c = jnp.full((LANES,), cvec[k], jnp.int32)
                e0 = plsc.load_gather(ref, [rows0, c])
                e1 = plsc.load_gather(ref, [rows1, c])
                stage[slot, k, pl.ds(0, LANES)] = e0
                stage[slot, k, pl.ds(LANES, LANES)] = e1
                pltpu.async_copy(stage.at[slot, k],
                                 emb.at[pl.ds(pvec[k] * EMBED_DIM,
                                              EMBED_DIM)],
                                 osems[slot])

    issue(jnp.int32(0), 0)
    issue(jnp.int32(1), 1)
    issue(jnp.int32(2), 2)

    def superstep(t, carry):
        for j in range(NSLOT):
            g = NSLOT * t + j
            drain(g, j)
            drain_out(g, j)
            compute(g, j)

            @pl.when(g < N_GROUPS - NSLOT)
            def _(j=j, g=g):
                issue(g + NSLOT, j)

        return carry

    n_steps = (N_GROUPS - 2) // NSLOT
    lax.fori_loop(0, n_steps, superstep, None)
    for g in range(NSLOT * n_steps, N_GROUPS):
        drain(jnp.int32(g), g % NSLOT)
        drain_out(jnp.int32(g), g % NSLOT)
        compute(jnp.int32(g), g % NSLOT)
    for slot in range(NSLOT):
        for stage, emb, osems in ((ustage, uembf_hbm, osemus),
                                  (istage, iembf_hbm, osemis)):
            for k in range(GSIZE):
                pltpu.make_async_copy(stage.at[slot, k],
                                      emb.at[pl.ds(0, EMBED_DIM)],
                                      osems[slot]).wait()


def _dot_body(uembf_hbm, iembf_hbm, out_hbm, uv, iv, acc_v):
    wid = lax.axis_index("s") * NUM_CORES + lax.axis_index("c")
    base = wid * B_PER_W

    pltpu.sync_copy(uembf_hbm.at[pl.ds(base * EMBED_DIM,
                                       B_PER_W * EMBED_DIM)], uv)
    pltpu.sync_copy(iembf_hbm.at[pl.ds(base * EMBED_DIM,
                                       B_PER_W * EMBED_DIM)], iv)

    lane = lax.iota(jnp.int32, LANES)

    def chunk(c, carry):
        accs = jnp.zeros((LANES,), jnp.float32)
        for r in range(LANES):
            rb = (c * LANES + r) * EMBED_DIM
            p = (uv[pl.ds(rb, LANES)] * iv[pl.ds(rb, LANES)]
                 + uv[pl.ds(rb + LANES, LANES)]
                 * iv[pl.ds(rb + LANES, LANES)])
            s = jnp.sum(p)
            accs = jnp.where(lane == r, jnp.broadcast_to(s, (LANES,)), accs)
        acc_v[pl.ds(c * LANES, LANES)] = accs
        return carry

    lax.fori_loop(0, B_PER_W // LANES, chunk, None)
    pltpu.sync_copy(acc_v, out_hbm.at[pl.ds(base, B_PER_W)])


def _prep(ids):
    iot = lax.iota(jnp.int32, BATCH)
    s, p = lax.sort_key_val(ids, iot)
    q = s >> 7
    newcol = jnp.concatenate(
        [jnp.ones((1,), jnp.int32), (q[1:] != q[:-1]).astype(jnp.int32)])
    flag = newcol | ((iot & (GSIZE - 1)) == 0).astype(jnp.int32)
    pos = iot & (GSIZE - 1)
    bidx = lax.cummax(jnp.where(flag != 0, pos, 0).reshape(-1, GSIZE),
                      axis=1).reshape(-1).astype(jnp.int32)
    return s, flag, bidx, p


@jax.jit
def kernel(user_ids, item_ids, user_table, item_table):
    su, fu, bu, pu = _prep(user_ids.astype(jnp.int32))
    si, fi, bi, pi = _prep(item_ids.astype(jnp.int32))

    mesh = plsc.VectorSubcoreMesh(core_axis_name="c", subcore_axis_name="s")
    params = pltpu.CompilerParams(needs_layout_passes=False)

    gather_k = functools.partial(
        pl.kernel,
        mesh=mesh,
        compiler_params=params,
        out_type=(
            jax.ShapeDtypeStruct((BATCH * EMBED_DIM,), jnp.float32),
            jax.ShapeDtypeStruct((BATCH * EMBED_DIM,), jnp.float32),
        ),
        scratch_types=(
            [pltpu.VMEM((IDX_PAD,), jnp.int32) for _ in range(8)]
            + [
                pltpu.VMEM((NSLOT, GSIZE, EMBED_DIM, 128), jnp.float32),
                pltpu.VMEM((NSLOT, GSIZE, EMBED_DIM, 128), jnp.float32),
                pltpu.VMEM((NSLOT, GSIZE, EMBED_DIM), jnp.float32),
                pltpu.VMEM((NSLOT, GSIZE, EMBED_DIM), jnp.float32),
            ]
            + [pltpu.SemaphoreType.DMA for _ in range(12)]
        ),
    )(_gather_body)
    uembf, iembf = gather_k(su, fu, bu, pu, si, fi, bi, pi,
                            user_table.T, item_table.T)

    dot_k = functools.partial(
        pl.kernel,
        mesh=mesh,
        compiler_params=params,
        out_type=jax.ShapeDtypeStruct((BATCH,), jnp.float32),
        scratch_types=[
            pltpu.VMEM((B_PER_W * EMBED_DIM,), jnp.float32),
            pltpu.VMEM((B_PER_W * EMBED_DIM,), jnp.float32),
            pltpu.VMEM((B_PER_W,), jnp.float32),
        ],
    )(_dot_body)
    return dot_k(uembf, iembf)
